# baseline (speedup 1.0000x reference)
import jax
import jax.numpy as jnp
from jax.experimental import pallas as pl

_TILE = 512


def _moe_body(x_ref, gates_ref, wg_ref, bg_ref, wflat_ref, be_ref, p_ref, s_ref, out_ref):
    xb = x_ref[...]
    g = jnp.dot(xb, wg_ref[...], preferred_element_type=jnp.float32)
    g = (g + bg_ref[...]) * gates_ref[...]
    y = jnp.dot(xb, wflat_ref[...], preferred_element_type=jnp.float32)
    ge = jnp.dot(g, p_ref[...], preferred_element_type=jnp.float32)
    out = jnp.dot(ge * y, s_ref[...], preferred_element_type=jnp.float32)
    out_ref[...] = out + jnp.dot(g, be_ref[...], preferred_element_type=jnp.float32)


def kernel(x, gates, W_gate, b_gate, W_experts, b_experts):
    B, D = x.shape
    E = gates.shape[1]
    O = W_experts.shape[2]
    w_flat = jnp.transpose(W_experts, (1, 0, 2)).reshape(D, E * O)
    bg2 = b_gate.reshape(1, E)
    eyeO = jnp.eye(O, dtype=jnp.float32)
    p_mat = jnp.repeat(jnp.eye(E, dtype=jnp.float32), O, axis=1)
    s_mat = jnp.tile(eyeO, (E, 1))
    tile = _TILE if B % _TILE == 0 else B
    grid = (B // tile,)
    return pl.pallas_call(
        _moe_body,
        grid=grid,
        in_specs=[
            pl.BlockSpec((tile, D), lambda i: (i, 0)),
            pl.BlockSpec((tile, E), lambda i: (i, 0)),
            pl.BlockSpec((D, E), lambda i: (0, 0)),
            pl.BlockSpec((1, E), lambda i: (0, 0)),
            pl.BlockSpec((D, E * O), lambda i: (0, 0)),
            pl.BlockSpec((E, O), lambda i: (0, 0)),
            pl.BlockSpec((E, E * O), lambda i: (0, 0)),
            pl.BlockSpec((E * O, O), lambda i: (0, 0)),
        ],
        out_specs=pl.BlockSpec((tile, O), lambda i: (i, 0)),
        out_shape=jax.ShapeDtypeStruct((B, O), jnp.float32),
    )(x, gates, W_gate, bg2, w_flat, b_experts, p_mat, s_mat)


# speedup vs baseline: 1.1138x; 1.1138x over previous
import jax
import jax.numpy as jnp
from jax.experimental import pallas as pl

_TILE = 1024


def _moe_body(x_ref, gates_ref, wg_ref, bg_ref, wflat_ref, be_ref, p_ref, s_ref, out_ref):
    xb = x_ref[...]
    g = jnp.dot(xb, wg_ref[...], preferred_element_type=jnp.float32)
    g = (g + bg_ref[...]) * gates_ref[...]
    y = jnp.dot(xb, wflat_ref[...], preferred_element_type=jnp.float32)
    ge = jnp.dot(g, p_ref[...], preferred_element_type=jnp.float32)
    out = jnp.dot(ge * y, s_ref[...], preferred_element_type=jnp.float32)
    out_ref[...] = out + jnp.dot(g, be_ref[...], preferred_element_type=jnp.float32)


def kernel(x, gates, W_gate, b_gate, W_experts, b_experts):
    B, D = x.shape
    E = gates.shape[1]
    O = W_experts.shape[2]
    w_flat = jnp.transpose(W_experts, (1, 0, 2)).reshape(D, E * O)
    bg2 = b_gate.reshape(1, E)
    eyeO = jnp.eye(O, dtype=jnp.float32)
    p_mat = jnp.repeat(jnp.eye(E, dtype=jnp.float32), O, axis=1)
    s_mat = jnp.tile(eyeO, (E, 1))
    tile = _TILE if B % _TILE == 0 else B
    grid = (B // tile,)
    return pl.pallas_call(
        _moe_body,
        grid=grid,
        in_specs=[
            pl.BlockSpec((tile, D), lambda i: (i, 0)),
            pl.BlockSpec((tile, E), lambda i: (i, 0)),
            pl.BlockSpec((D, E), lambda i: (0, 0)),
            pl.BlockSpec((1, E), lambda i: (0, 0)),
            pl.BlockSpec((D, E * O), lambda i: (0, 0)),
            pl.BlockSpec((E, O), lambda i: (0, 0)),
            pl.BlockSpec((E, E * O), lambda i: (0, 0)),
            pl.BlockSpec((E * O, O), lambda i: (0, 0)),
        ],
        out_specs=pl.BlockSpec((tile, O), lambda i: (i, 0)),
        out_shape=jax.ShapeDtypeStruct((B, O), jnp.float32),
    )(x, gates, W_gate, bg2, w_flat, b_experts, p_mat, s_mat)


# speedup vs baseline: 1.1197x; 1.0053x over previous
import jax
import jax.numpy as jnp
from jax.experimental import pallas as pl

_TILE = 2048


def _moe_body(x_ref, gates_ref, wg_ref, bg_ref, wflat_ref, be_ref, p_ref, s_ref, out_ref):
    xb = x_ref[...]
    g = jnp.dot(xb, wg_ref[...], preferred_element_type=jnp.float32)
    g = (g + bg_ref[...]) * gates_ref[...]
    y = jnp.dot(xb, wflat_ref[...], preferred_element_type=jnp.float32)
    ge = jnp.dot(g, p_ref[...], preferred_element_type=jnp.float32)
    out = jnp.dot(ge * y, s_ref[...], preferred_element_type=jnp.float32)
    out_ref[...] = out + jnp.dot(g, be_ref[...], preferred_element_type=jnp.float32)


def kernel(x, gates, W_gate, b_gate, W_experts, b_experts):
    B, D = x.shape
    E = gates.shape[1]
    O = W_experts.shape[2]
    w_flat = jnp.transpose(W_experts, (1, 0, 2)).reshape(D, E * O)
    bg2 = b_gate.reshape(1, E)
    eyeO = jnp.eye(O, dtype=jnp.float32)
    p_mat = jnp.repeat(jnp.eye(E, dtype=jnp.float32), O, axis=1)
    s_mat = jnp.tile(eyeO, (E, 1))
    tile = _TILE if B % _TILE == 0 else B
    grid = (B // tile,)
    return pl.pallas_call(
        _moe_body,
        grid=grid,
        in_specs=[
            pl.BlockSpec((tile, D), lambda i: (i, 0)),
            pl.BlockSpec((tile, E), lambda i: (i, 0)),
            pl.BlockSpec((D, E), lambda i: (0, 0)),
            pl.BlockSpec((1, E), lambda i: (0, 0)),
            pl.BlockSpec((D, E * O), lambda i: (0, 0)),
            pl.BlockSpec((E, O), lambda i: (0, 0)),
            pl.BlockSpec((E, E * O), lambda i: (0, 0)),
            pl.BlockSpec((E * O, O), lambda i: (0, 0)),
        ],
        out_specs=pl.BlockSpec((tile, O), lambda i: (i, 0)),
        out_shape=jax.ShapeDtypeStruct((B, O), jnp.float32),
    )(x, gates, W_gate, bg2, w_flat, b_experts, p_mat, s_mat)
